# baseline (speedup 1.0000x reference)
import jax
import jax.numpy as jnp
from jax.experimental import pallas as pl

_HIDDEN = 768
_RANK = _HIDDEN // 4
_THRESH = 0.5
_N = 4 * 8192
_B = 4096
_S = 4
_BS = _B // _S
_NB = _N // _B
_REG = _N // _S


def _gate_kernel(*refs):
    x_refs = refs[:_S]
    wt = refs[_S][...]
    mask_refs = refs[_S + 1:_S + 1 + _S]
    score_refs = refs[_S + 1 + _S:]
    for s in range(_S):
        x = x_refs[s][...]
        h = jnp.dot(x, wt, preferred_element_type=jnp.float32)
        s2 = jnp.sum(h * h, axis=-1, keepdims=True)
        score = jnp.sqrt(s2)
        m = score >= _THRESH
        mask_refs[s][...] = m.astype(jnp.float32)
        score_refs[s][...] = jnp.where(m, score, -jnp.inf)


def _stream_index_map(s):
    off = s * (_REG // _BS)
    return lambda i: (off + i, 0)


def kernel(x, W):
    xf = x.reshape(_N, _HIDDEN)
    wt = W.T
    in_specs = [pl.BlockSpec((_BS, _HIDDEN), _stream_index_map(s))
                for s in range(_S)]
    in_specs.append(pl.BlockSpec((_HIDDEN, _RANK), lambda i: (0, 0)))
    out_specs = [pl.BlockSpec((_BS, 1), lambda i: (i, 0))
                 for _ in range(2 * _S)]
    out_shape = [jax.ShapeDtypeStruct((_REG, 1), jnp.float32)
                 for _ in range(2 * _S)]
    outs = pl.pallas_call(
        _gate_kernel,
        grid=(_NB,),
        in_specs=in_specs,
        out_specs=out_specs,
        out_shape=out_shape,
    )(*([xf] * _S), wt)
    mask_f = jnp.concatenate(outs[:_S], axis=0)
    score = jnp.concatenate(outs[_S:], axis=0)
    lead = x.shape[:-1]
    return mask_f.reshape(lead).astype(jnp.bool_), score.reshape(lead)


# speedup vs baseline: 1.0717x; 1.0717x over previous
import jax
import jax.numpy as jnp
from jax.experimental import pallas as pl
from jax.experimental.pallas import tpu as pltpu

_HIDDEN = 768
_RANK = _HIDDEN // 4
_THRESH = 0.5
_N = 4 * 8192
_CH = 1024
_NCH = _N // _CH
_NBUF = 4


def _start_copy(x_hbm, xbuf, sems, chunk, slot):
    pltpu.make_async_copy(
        x_hbm.at[pl.ds(chunk * _CH, _CH), :],
        xbuf.at[slot],
        sems.at[slot],
    ).start()


def _gate_kernel(x_hbm, wt_ref, mask_ref, score_ref, xbuf, sems):
    i = pl.program_id(0)

    @pl.when(i == 0)
    def _():
        for b in range(_NBUF - 1):
            _start_copy(x_hbm, xbuf, sems, b, b)

    nxt = i + _NBUF - 1

    @pl.when(nxt < _NCH)
    def _():
        _start_copy(x_hbm, xbuf, sems, nxt, jax.lax.rem(nxt, _NBUF))

    slot = jax.lax.rem(i, _NBUF)
    pltpu.make_async_copy(
        x_hbm.at[pl.ds(i * _CH, _CH), :],
        xbuf.at[slot],
        sems.at[slot],
    ).wait()

    x = xbuf[slot]
    h = jnp.dot(x, wt_ref[...], preferred_element_type=jnp.float32)
    s2 = jnp.sum(h * h, axis=-1, keepdims=True)
    score = jnp.sqrt(s2)
    m = score >= _THRESH
    mask_ref[...] = m.astype(jnp.float32)
    score_ref[...] = jnp.where(m, score, -jnp.inf)


def kernel(x, W):
    xf = x.reshape(_N, _HIDDEN)
    wt = W.T
    mask_f, score = pl.pallas_call(
        _gate_kernel,
        grid=(_NCH,),
        in_specs=[
            pl.BlockSpec(memory_space=pl.ANY),
            pl.BlockSpec((_HIDDEN, _RANK), lambda i: (0, 0)),
        ],
        out_specs=[
            pl.BlockSpec((_CH, 1), lambda i: (i, 0)),
            pl.BlockSpec((_CH, 1), lambda i: (i, 0)),
        ],
        out_shape=[
            jax.ShapeDtypeStruct((_N, 1), jnp.float32),
            jax.ShapeDtypeStruct((_N, 1), jnp.float32),
        ],
        scratch_shapes=[
            pltpu.VMEM((_NBUF, _CH, _HIDDEN), jnp.float32),
            pltpu.SemaphoreType.DMA((_NBUF,)),
        ],
    )(xf, wt)
    lead = x.shape[:-1]
    return mask_f.reshape(lead).astype(jnp.bool_), score.reshape(lead)
